# baseline (speedup 1.0000x reference)
import functools

import jax
import jax.numpy as jnp
from jax import lax
from jax.experimental import pallas as pl
from jax.experimental.pallas import tpu as pltpu
from jax.experimental.pallas import tpu_sc as plsc

_VOCAB = 1000000
_D = 16
_B = 4096
_L = 200

_NC = 2
_NS = 16
_NW = _NC * _NS

_CHUNK = 100
_HALVES = _L // _CHUNK
_ROWS_PER_W = _B // _NW
_IDX_ROWS = _ROWS_PER_W * _HALVES
_NBUF = 8

_BLK = 128
_NBLK_FULL = _VOCAB // _BLK
_TAIL = _VOCAB - _NBLK_FULL * _BLK
_CBLK = 4
_CROWS = _CBLK * _BLK
_NCH = _NBLK_FULL // _CBLK


def _wid():
    return lax.axis_index("s") * _NC + lax.axis_index("c")


_NRB = 2


def _sc_relayout(tblT_hbm, tail_hbm, out_hbm, *rest):
    in_v = rest[:_NRB]
    tr_v = rest[_NRB : 2 * _NRB]
    tail_v = rest[2 * _NRB]
    isem = rest[2 * _NRB + 1]
    osem = rest[2 * _NRB + 2]
    wid = _wid()
    rows16 = jnp.arange(_D, dtype=jnp.int32)

    nfull = jnp.where(wid < _NBLK_FULL - 244 * _NW, 245, 244)

    def blk_id(k):
        return wid + _NW * k

    def issue_in(k, e):
        pltpu.async_copy(
            tblT_hbm.at[:, pl.ds(blk_id(k) * _BLK, _BLK)], in_v[e], isem.at[e]
        )

    scat16 = rows16 * _D

    def transpose_block(e):
        for c in range(0, _D, 2):
            vs = [
                (in_v[e][cc, pl.ds(m * _D, _D)], m * _D * _D + cc)
                for cc in (c, c + 1)
                for m in range(_BLK // _D)
            ]
            for v, off in vs:
                plsc.store_scatter(tr_v[e], [scat16 + off], v)

    for e in range(_NRB):
        issue_in(e, e)

    @pl.loop(0, 246, step=_NRB)
    def _(kbase):
        for e in range(_NRB):
            k = kbase + e

            @pl.when(k < nfull)
            def _():
                pltpu.make_async_copy(
                    tblT_hbm.at[:, pl.ds(0, _BLK)], in_v[e], isem.at[e]
                ).wait()

                @pl.when(k >= _NRB)
                def _():
                    pltpu.make_async_copy(
                        tr_v[e], out_hbm.at[pl.ds(0, _BLK * _D)], osem.at[e]
                    ).wait()

                transpose_block(e)

                @pl.when(k + _NRB < nfull)
                def _():
                    issue_in(k + _NRB, e)

                pltpu.async_copy(
                    tr_v[e],
                    out_hbm.at[pl.ds(blk_id(k) * _BLK * _D, _BLK * _D)],
                    osem.at[e],
                )

    for e in range(_NRB):
        pltpu.make_async_copy(
            tr_v[e], out_hbm.at[pl.ds(0, _BLK * _D)], osem.at[e]
        ).wait()

    @pl.when(wid == 0)
    def _():
        pltpu.sync_copy(tail_hbm, tail_v)
        pltpu.sync_copy(
            tail_v, out_hbm.at[pl.ds(_NBLK_FULL * _BLK * _D, _TAIL * _D)]
        )


def _tree_sum(buf):
    vals = [buf[l, :] for l in range(_CHUNK)]
    while len(vals) > 1:
        nxt = [vals[i] + vals[i + 1] for i in range(0, len(vals) - 1, 2)]
        if len(vals) % 2:
            nxt.append(vals[-1])
        vals = nxt
    return vals[0]


def _sc_gather(idx_hbm, table_hbm, out_hbm, idx_v, out_v, *rest):
    bufs = rest[:_NBUF]
    sems = rest[_NBUF]
    wid = _wid()

    pltpu.sync_copy(idx_hbm.at[pl.ds(wid * _IDX_ROWS, _IDX_ROWS), :], idx_v)

    for b in range(_NBUF):
        pltpu.async_copy(table_hbm.at[idx_v.at[b]], bufs[b], sems.at[b])

    @pl.loop(0, _IDX_ROWS, step=_NBUF)
    def _(j):
        for p in range(_NBUF // 2):
            sums = []
            for b in (2 * p, 2 * p + 1):
                h = j + b
                pltpu.make_async_copy(
                    table_hbm.at[idx_v.at[0]], bufs[b], sems.at[b]
                ).wait()
                sums.append(_tree_sum(bufs[b]))

                @pl.when(h + _NBUF < _IDX_ROWS)
                def _():
                    pltpu.async_copy(
                        table_hbm.at[idx_v.at[h + _NBUF]], bufs[b], sems.at[b]
                    )

            r = j // _HALVES + p
            out_v[r, :] = (sums[0] + sums[1]) * (1.0 / _L)

    pltpu.sync_copy(out_v, out_hbm.at[pl.ds(wid * _ROWS_PER_W, _ROWS_PER_W), :])


_SC_MESH = dict(
    mesh=plsc.VectorSubcoreMesh(core_axis_name="c", subcore_axis_name="s"),
)


@jax.jit
def kernel(indices, table):
    relayout = functools.partial(
        pl.kernel,
        out_type=jax.ShapeDtypeStruct((_VOCAB * _D,), jnp.float32),
        compiler_params=pltpu.CompilerParams(needs_layout_passes=False),
        scratch_types=[pltpu.VMEM((_D, _BLK), jnp.float32) for _ in range(_NRB)]
        + [pltpu.VMEM((_BLK * _D,), jnp.float32) for _ in range(_NRB)]
        + [
            pltpu.VMEM((_TAIL * _D,), jnp.float32),
            pltpu.SemaphoreType.DMA((_NRB,)),
            pltpu.SemaphoreType.DMA((_NRB,)),
        ],
        **_SC_MESH,
    )(_sc_relayout)
    tail_flat = table[_NBLK_FULL * _BLK :].reshape(_TAIL * _D)
    flat = relayout(table.T, tail_flat)
    tbl_rm = flat.reshape(_VOCAB, _D)

    idx2d = indices.reshape(_B * _HALVES, _CHUNK)
    gather = functools.partial(
        pl.kernel,
        out_type=jax.ShapeDtypeStruct((_B, _D), jnp.float32),
        compiler_params=pltpu.CompilerParams(use_tc_tiling_on_sc=False),
        scratch_types=[
            pltpu.VMEM((_IDX_ROWS, _CHUNK), jnp.int32),
            pltpu.VMEM((_ROWS_PER_W, _D), jnp.float32),
        ]
        + [pltpu.VMEM((_CHUNK, _D), jnp.float32) for _ in range(_NBUF)]
        + [pltpu.SemaphoreType.DMA((_NBUF,))],
        **_SC_MESH,
    )(_sc_gather)
    return gather(idx2d, tbl_rm)


# speedup vs baseline: 1.0906x; 1.0906x over previous
import functools

import jax
import jax.numpy as jnp
from jax import lax
from jax.experimental import pallas as pl
from jax.experimental.pallas import tpu as pltpu
from jax.experimental.pallas import tpu_sc as plsc

_VOCAB = 1000000
_D = 16
_B = 4096
_L = 200

_NC = 2
_NS = 16
_NW = _NC * _NS

_CHUNK = 100
_HALVES = _L // _CHUNK
_ROWS_PER_W = _B // _NW
_IDX_ROWS = _ROWS_PER_W * _HALVES
_NBUF = 8

_BLK = 128
_NBLK_FULL = _VOCAB // _BLK
_TAIL = _VOCAB - _NBLK_FULL * _BLK
_CBLK = 4
_CROWS = _CBLK * _BLK
_NCH = _NBLK_FULL // _CBLK


def _wid():
    return lax.axis_index("s") * _NC + lax.axis_index("c")


_NRB = 4


def _sc_relayout(tblT_hbm, tail_hbm, out_hbm, *rest):
    in_v = rest[:_NRB]
    tr_v = rest[_NRB : 2 * _NRB]
    tail_v = rest[2 * _NRB]
    isem = rest[2 * _NRB + 1]
    osem = rest[2 * _NRB + 2]
    wid = _wid()
    rows16 = jnp.arange(_D, dtype=jnp.int32)

    nfull = jnp.where(wid < _NBLK_FULL - 244 * _NW, 245, 244)

    def blk_id(k):
        return wid + _NW * k

    def issue_in(k, e):
        pltpu.async_copy(
            tblT_hbm.at[:, pl.ds(blk_id(k) * _BLK, _BLK)], in_v[e], isem.at[e]
        )

    scat16 = rows16 * _D

    def transpose_block(e):
        for c in range(_D):
            for m in range(_BLK // _D):
                v = in_v[e][c, pl.ds(m * _D, _D)]
                plsc.store_scatter(tr_v[e], [scat16 + (m * _D * _D + c)], v)

    for e in range(_NRB):
        issue_in(e, e)

    @pl.loop(0, 248, step=_NRB)
    def _(kbase):
        for e in range(_NRB):
            k = kbase + e

            @pl.when(k < nfull)
            def _():
                pltpu.make_async_copy(
                    tblT_hbm.at[:, pl.ds(0, _BLK)], in_v[e], isem.at[e]
                ).wait()

                @pl.when(k >= _NRB)
                def _():
                    pltpu.make_async_copy(
                        tr_v[e], out_hbm.at[pl.ds(0, _BLK * _D)], osem.at[e]
                    ).wait()

                transpose_block(e)

                @pl.when(k + _NRB < nfull)
                def _():
                    issue_in(k + _NRB, e)

                pltpu.async_copy(
                    tr_v[e],
                    out_hbm.at[pl.ds(blk_id(k) * _BLK * _D, _BLK * _D)],
                    osem.at[e],
                )

    for e in range(_NRB):
        pltpu.make_async_copy(
            tr_v[e], out_hbm.at[pl.ds(0, _BLK * _D)], osem.at[e]
        ).wait()

    @pl.when(wid == 0)
    def _():
        pltpu.sync_copy(tail_hbm, tail_v)
        pltpu.sync_copy(
            tail_v, out_hbm.at[pl.ds(_NBLK_FULL * _BLK * _D, _TAIL * _D)]
        )


def _tree_sum(buf):
    vals = [buf[l, :] for l in range(_CHUNK)]
    while len(vals) > 1:
        nxt = [vals[i] + vals[i + 1] for i in range(0, len(vals) - 1, 2)]
        if len(vals) % 2:
            nxt.append(vals[-1])
        vals = nxt
    return vals[0]


def _sc_gather(idx_hbm, table_hbm, out_hbm, idx_v, out_v, *rest):
    bufs = rest[:_NBUF]
    sems = rest[_NBUF]
    wid = _wid()

    pltpu.sync_copy(idx_hbm.at[pl.ds(wid * _IDX_ROWS, _IDX_ROWS), :], idx_v)

    for b in range(_NBUF):
        pltpu.async_copy(table_hbm.at[idx_v.at[b]], bufs[b], sems.at[b])

    @pl.loop(0, _IDX_ROWS, step=_NBUF)
    def _(j):
        for p in range(_NBUF // 2):
            sums = []
            for b in (2 * p, 2 * p + 1):
                h = j + b
                pltpu.make_async_copy(
                    table_hbm.at[idx_v.at[0]], bufs[b], sems.at[b]
                ).wait()
                sums.append(_tree_sum(bufs[b]))

                @pl.when(h + _NBUF < _IDX_ROWS)
                def _():
                    pltpu.async_copy(
                        table_hbm.at[idx_v.at[h + _NBUF]], bufs[b], sems.at[b]
                    )

            r = j // _HALVES + p
            out_v[r, :] = (sums[0] + sums[1]) * (1.0 / _L)

    pltpu.sync_copy(out_v, out_hbm.at[pl.ds(wid * _ROWS_PER_W, _ROWS_PER_W), :])


_SC_MESH = dict(
    mesh=plsc.VectorSubcoreMesh(core_axis_name="c", subcore_axis_name="s"),
)


@jax.jit
def kernel(indices, table):
    relayout = functools.partial(
        pl.kernel,
        out_type=jax.ShapeDtypeStruct((_VOCAB * _D,), jnp.float32),
        compiler_params=pltpu.CompilerParams(needs_layout_passes=False),
        scratch_types=[pltpu.VMEM((_D, _BLK), jnp.float32) for _ in range(_NRB)]
        + [pltpu.VMEM((_BLK * _D,), jnp.float32) for _ in range(_NRB)]
        + [
            pltpu.VMEM((_TAIL * _D,), jnp.float32),
            pltpu.SemaphoreType.DMA((_NRB,)),
            pltpu.SemaphoreType.DMA((_NRB,)),
        ],
        **_SC_MESH,
    )(_sc_relayout)
    tail_flat = table[_NBLK_FULL * _BLK :].reshape(_TAIL * _D)
    flat = relayout(table.T, tail_flat)
    tbl_rm = flat.reshape(_VOCAB, _D)

    idx2d = indices.reshape(_B * _HALVES, _CHUNK)
    gather = functools.partial(
        pl.kernel,
        out_type=jax.ShapeDtypeStruct((_B, _D), jnp.float32),
        compiler_params=pltpu.CompilerParams(use_tc_tiling_on_sc=False),
        scratch_types=[
            pltpu.VMEM((_IDX_ROWS, _CHUNK), jnp.int32),
            pltpu.VMEM((_ROWS_PER_W, _D), jnp.float32),
        ]
        + [pltpu.VMEM((_CHUNK, _D), jnp.float32) for _ in range(_NBUF)]
        + [pltpu.SemaphoreType.DMA((_NBUF,))],
        **_SC_MESH,
    )(_sc_gather)
    return gather(idx2d, tbl_rm)
